# baseline (speedup 1.0000x reference)
import functools

import jax
import jax.numpy as jnp
from jax import lax
from jax.experimental import pallas as pl
from jax.experimental.pallas import tpu as pltpu
from jax.experimental.pallas import tpu_sc as plsc

BATCH = 128
TABLE = 1001
LANES = 16
NTILES = BATCH // LANES


def _body(t_hbm, flex_hbm, sf_hbm, si_hbm, out_hbm,
          t_v, sf_rows, si_rows, out_v, flex_v, sem, sem2):
    cid = lax.axis_index("c")
    sid = lax.axis_index("s")

    @pl.when((cid == 0) & (sid < NTILES))
    def _():
        base = sid * LANES
        c0 = pltpu.async_copy(flex_hbm, flex_v.at[pl.ds(0, 1)], sem2)
        pltpu.sync_copy(t_hbm.at[pl.ds(base, LANES)], t_v)
        g1 = pltpu.async_copy(sf_hbm.at[t_v], sf_rows, sem)
        g2 = pltpu.async_copy(si_hbm.at[t_v], si_rows, sem)
        c0.wait()
        dn = lax.GatherDimensionNumbers(
            offset_dims=(), collapsed_slice_dims=(0,), start_index_map=(0,))
        flex = lax.gather(
            flex_v[...], jnp.zeros((LANES, 1), jnp.int32), dn, slice_sizes=(1,),
            mode=lax.GatherScatterMode.PROMISE_IN_BOUNDS)
        omf = 1.0 - flex
        g1.wait()
        g2.wait()
        out_v[...] = sf_rows[...] * flex + si_rows[...] * omf
        pltpu.sync_copy(out_v, out_hbm.at[pl.ds(base, LANES)])


@jax.jit
def kernel(t, flexibility, sigmas_flex, sigmas_inflex):
    t32 = t.astype(jnp.int32)
    mesh = plsc.VectorSubcoreMesh(core_axis_name="c", subcore_axis_name="s",
                                  num_cores=1)
    f = functools.partial(
        pl.kernel,
        out_type=jax.ShapeDtypeStruct((BATCH,), jnp.float32),
        mesh=mesh,
        scratch_types=[
            pltpu.VMEM((LANES,), jnp.int32),
            pltpu.VMEM((LANES,), jnp.float32),
            pltpu.VMEM((LANES,), jnp.float32),
            pltpu.VMEM((LANES,), jnp.float32),
            pltpu.VMEM((LANES,), jnp.float32),
            pltpu.SemaphoreType.DMA,
            pltpu.SemaphoreType.DMA,
        ],
    )(_body)
    return f(t32, flexibility, sigmas_flex, sigmas_inflex)


# speedup vs baseline: 1.0072x; 1.0072x over previous
import functools

import jax
import jax.numpy as jnp
from jax import lax
from jax.experimental import pallas as pl
from jax.experimental.pallas import tpu as pltpu
from jax.experimental.pallas import tpu_sc as plsc

BATCH = 128
TABLE = 1001
LANES = 16
NTILES = BATCH // LANES


def _body(t_hbm, flex_hbm, sf_hbm, si_hbm, out_hbm,
          t_v, sf_rows, si_rows, flex_v, sem, sem2):
    sid = lax.axis_index("s")

    @pl.when(sid < NTILES)
    def _():
        base = sid * LANES
        c0 = pltpu.async_copy(flex_hbm, flex_v.at[pl.ds(0, 1)], sem2)
        pltpu.sync_copy(t_hbm.at[pl.ds(base, LANES)], t_v)
        g1 = pltpu.async_copy(sf_hbm.at[t_v], sf_rows, sem)
        g2 = pltpu.async_copy(si_hbm.at[t_v], si_rows, sem)
        c0.wait()
        dn = lax.GatherDimensionNumbers(
            offset_dims=(), collapsed_slice_dims=(0,), start_index_map=(0,))
        flex = lax.gather(
            flex_v[...], jnp.zeros((LANES, 1), jnp.int32), dn, slice_sizes=(1,),
            mode=lax.GatherScatterMode.PROMISE_IN_BOUNDS)
        omf = 1.0 - flex
        g1.wait()
        g2.wait()
        sf_rows[...] = sf_rows[...] * flex + si_rows[...] * omf
        pltpu.sync_copy(sf_rows, out_hbm.at[pl.ds(base, LANES)])


@jax.jit
def kernel(t, flexibility, sigmas_flex, sigmas_inflex):
    t32 = t.astype(jnp.int32)
    mesh = plsc.VectorSubcoreMesh(core_axis_name="c", subcore_axis_name="s",
                                  num_cores=1)
    f = functools.partial(
        pl.kernel,
        out_type=jax.ShapeDtypeStruct((BATCH,), jnp.float32),
        mesh=mesh,
        scratch_types=[
            pltpu.VMEM((LANES,), jnp.int32),
            pltpu.VMEM((LANES,), jnp.float32),
            pltpu.VMEM((LANES,), jnp.float32),
            pltpu.VMEM((LANES,), jnp.float32),
            pltpu.SemaphoreType.DMA,
            pltpu.SemaphoreType.DMA,
        ],
    )(_body)
    return f(t32, flexibility, sigmas_flex, sigmas_inflex)


# speedup vs baseline: 1.0108x; 1.0035x over previous
import functools

import jax
import jax.numpy as jnp
from jax import lax
from jax.experimental import pallas as pl
from jax.experimental.pallas import tpu as pltpu
from jax.experimental.pallas import tpu_sc as plsc

BATCH = 128
TABLE = 1001
LANES = 16
NTILES = BATCH // LANES


def _body(t_hbm, flex_hbm, sf_hbm, si_hbm, out_hbm,
          t_v, sf_rows, si_rows, flex_v, sem, sem2):
    sid = lax.axis_index("s")

    @pl.when(sid < NTILES)
    def _():
        base = sid * LANES
        c0 = pltpu.async_copy(flex_hbm, flex_v.at[pl.ds(0, 1)], sem2)
        pltpu.sync_copy(t_hbm.at[pl.ds(base, LANES)], t_v)
        g1 = pltpu.async_copy(sf_hbm.at[t_v], sf_rows, sem)
        g2 = pltpu.async_copy(si_hbm.at[t_v], si_rows, sem)
        c0.wait()
        dn = lax.GatherDimensionNumbers(
            offset_dims=(), collapsed_slice_dims=(0,), start_index_map=(0,))
        flex = lax.gather(
            flex_v[...], jnp.zeros((LANES, 1), jnp.int32), dn, slice_sizes=(1,),
            mode=lax.GatherScatterMode.PROMISE_IN_BOUNDS)
        omf = 1.0 - flex
        g1.wait()
        g2.wait()
        sf_rows[...] = sf_rows[...] * flex + si_rows[...] * omf
        pltpu.sync_copy(sf_rows, out_hbm.at[pl.ds(base, LANES)])


@jax.jit
def kernel(t, flexibility, sigmas_flex, sigmas_inflex):
    t32 = t.astype(jnp.int32)
    mesh = plsc.VectorSubcoreMesh(core_axis_name="c", subcore_axis_name="s",
                                  num_cores=1, num_subcores=8)
    f = functools.partial(
        pl.kernel,
        out_type=jax.ShapeDtypeStruct((BATCH,), jnp.float32),
        mesh=mesh,
        scratch_types=[
            pltpu.VMEM((LANES,), jnp.int32),
            pltpu.VMEM((LANES,), jnp.float32),
            pltpu.VMEM((LANES,), jnp.float32),
            pltpu.VMEM((LANES,), jnp.float32),
            pltpu.SemaphoreType.DMA,
            pltpu.SemaphoreType.DMA,
        ],
    )(_body)
    return f(t32, flexibility, sigmas_flex, sigmas_inflex)
